# baseline (speedup 1.0000x reference)
import jax
import jax.numpy as jnp
from jax.experimental import pallas as pl
from jax.experimental.pallas import tpu as pltpu

_B, _N, _NBH = 8, 512, 32
_AB, _SB, _NF = 256, 64, 256
_BN = 64
_NBLK = _N // _BN
_E = _BN * _NBH

_BF = jnp.bfloat16


def _ssp(v):
    return jnp.logaddexp(v, 0.0) - jnp.log(2.0)


def _block_kernel(x_ref, nbr_ref, mask_ref, f_ref,
                  W1_ref, b1_ref, W2_ref, b2_ref,
                  Wi_ref, Wf_ref, bf_ref, Wd_ref, bd_ref,
                  out_ref, y_scr):
    nb = pl.program_id(1)

    @pl.when(nb == 0)
    def _():
        y_scr[:] = jnp.dot(x_ref[0], Wi_ref[:],
                           preferred_element_type=jnp.float32).astype(_BF)

    h = _ssp(jnp.dot(f_ref[0, 0], W1_ref[:],
                     preferred_element_type=jnp.float32) + b1_ref[:])
    wfilt = jnp.dot(h.astype(_BF), W2_ref[:],
                    preferred_element_type=jnp.float32) + b2_ref[:]
    wfilt = wfilt * mask_ref[0, 0]

    idx = nbr_ref[0, 0]
    onehot = (idx == jax.lax.broadcasted_iota(jnp.int32, (_E, _N), 1)
              ).astype(_BF)
    y_nbh = jnp.dot(onehot, y_scr[:], preferred_element_type=jnp.float32)

    agg = (y_nbh * wfilt).reshape(_BN, _NBH, _NF).sum(axis=1)
    v = _ssp(jnp.dot(agg.astype(_BF), Wf_ref[:],
                     preferred_element_type=jnp.float32) + bf_ref[:])
    out_ref[0] = jnp.dot(v.astype(_BF), Wd_ref[:],
                         preferred_element_type=jnp.float32) + bd_ref[:]


def kernel(x, r_ij, neighbors, neighbor_mask, f_ij,
           W1, b1, W2, b2, Wi, Wf, bf, Wd, bd):
    del r_ij
    grid = (_B, _NBLK)
    full = lambda shape: pl.BlockSpec(shape, lambda b, nb: (0,) * len(shape))

    nbr_r = neighbors.reshape(_B, _NBLK, _E, 1)
    mask_r = neighbor_mask.reshape(_B, _NBLK, _E, 1)
    f_r = f_ij.reshape(_B, _NBLK, _E, _SB).astype(_BF)

    out = pl.pallas_call(
        _block_kernel,
        grid=grid,
        in_specs=[
            pl.BlockSpec((1, _N, _AB), lambda b, nb: (b, 0, 0)),
            pl.BlockSpec((1, 1, _E, 1), lambda b, nb: (b, nb, 0, 0)),
            pl.BlockSpec((1, 1, _E, 1), lambda b, nb: (b, nb, 0, 0)),
            pl.BlockSpec((1, 1, _E, _SB), lambda b, nb: (b, nb, 0, 0)),
            full((_SB, _NF)),
            full((1, _NF)),
            full((_NF, _NF)),
            full((1, _NF)),
            full((_AB, _NF)),
            full((_NF, _AB)),
            full((1, _AB)),
            full((_AB, _AB)),
            full((1, _AB)),
        ],
        out_specs=pl.BlockSpec((1, _BN, _AB), lambda b, nb: (b, nb, 0)),
        out_shape=jax.ShapeDtypeStruct((_B, _N, _AB), jnp.float32),
        scratch_shapes=[pltpu.VMEM((_N, _NF), _BF)],
        compiler_params=pltpu.CompilerParams(
            dimension_semantics=("parallel", "arbitrary"),
        ),
    )(x.astype(_BF), nbr_r, mask_r, f_r,
      W1.astype(_BF), b1.reshape(1, _NF), W2.astype(_BF), b2.reshape(1, _NF),
      Wi.astype(_BF), Wf.astype(_BF), bf.reshape(1, _AB),
      Wd.astype(_BF), bd.reshape(1, _AB))
    return out


# speedup vs baseline: 1.0418x; 1.0418x over previous
import jax
import jax.numpy as jnp
from jax.experimental import pallas as pl
from jax.experimental.pallas import tpu as pltpu

_B, _N, _NBH = 8, 512, 32
_AB, _SB, _NF = 256, 64, 256
_BN = 64
_NBLK = _N // _BN
_E = _BN * _NBH

_BF = jnp.bfloat16


def _ssp(v):
    return jnp.logaddexp(v, 0.0) - jnp.log(2.0)


def _y_kernel(x_ref, Wi_ref, y_ref):
    y_ref[0] = jnp.dot(x_ref[0].astype(_BF), Wi_ref[:],
                       preferred_element_type=jnp.float32).astype(_BF)


def _block_kernel(y_ref, nbr_ref, mask_ref, f_ref,
                  W1_ref, b1_ref, W2_ref, b2_ref,
                  Wf_ref, bf_ref, Wd_ref, bd_ref,
                  out_ref):
    h = _ssp(jnp.dot(f_ref[0, 0].astype(_BF), W1_ref[:],
                     preferred_element_type=jnp.float32) + b1_ref[:])
    wfilt = jnp.dot(h.astype(_BF), W2_ref[:],
                    preferred_element_type=jnp.float32) + b2_ref[:]
    wfilt = wfilt * mask_ref[0, 0]

    idx = nbr_ref[0, 0]
    onehot = (idx == jax.lax.broadcasted_iota(jnp.int32, (_E, _N), 1)
              ).astype(_BF)
    y_nbh = jnp.dot(onehot, y_ref[0], preferred_element_type=jnp.float32)

    agg = (y_nbh * wfilt).reshape(_BN, _NBH, _NF).sum(axis=1)
    v = _ssp(jnp.dot(agg.astype(_BF), Wf_ref[:],
                     preferred_element_type=jnp.float32) + bf_ref[:])
    out_ref[0] = jnp.dot(v.astype(_BF), Wd_ref[:],
                         preferred_element_type=jnp.float32) + bd_ref[:]


def kernel(x, r_ij, neighbors, neighbor_mask, f_ij,
           W1, b1, W2, b2, Wi, Wf, bf, Wd, bd):
    del r_ij

    y = pl.pallas_call(
        _y_kernel,
        grid=(_B,),
        in_specs=[
            pl.BlockSpec((1, _N, _AB), lambda b: (b, 0, 0)),
            pl.BlockSpec((_AB, _NF), lambda b: (0, 0)),
        ],
        out_specs=pl.BlockSpec((1, _N, _NF), lambda b: (b, 0, 0)),
        out_shape=jax.ShapeDtypeStruct((_B, _N, _NF), _BF),
        compiler_params=pltpu.CompilerParams(
            dimension_semantics=("parallel",),
        ),
    )(x, Wi.astype(_BF))

    nbr_r = neighbors.reshape(_B, _NBLK, _E, 1)
    mask_r = neighbor_mask.reshape(_B, _NBLK, _E, 1)
    f_r = f_ij.reshape(_B, _NBLK, _E, _SB)
    full = lambda shape: pl.BlockSpec(shape, lambda b, nb: (0,) * len(shape))

    out = pl.pallas_call(
        _block_kernel,
        grid=(_B, _NBLK),
        in_specs=[
            pl.BlockSpec((1, _N, _NF), lambda b, nb: (b, 0, 0)),
            pl.BlockSpec((1, 1, _E, 1), lambda b, nb: (b, nb, 0, 0)),
            pl.BlockSpec((1, 1, _E, 1), lambda b, nb: (b, nb, 0, 0)),
            pl.BlockSpec((1, 1, _E, _SB), lambda b, nb: (b, nb, 0, 0)),
            full((_SB, _NF)),
            full((1, _NF)),
            full((_NF, _NF)),
            full((1, _NF)),
            full((_NF, _AB)),
            full((1, _AB)),
            full((_AB, _AB)),
            full((1, _AB)),
        ],
        out_specs=pl.BlockSpec((1, _BN, _AB), lambda b, nb: (b, nb, 0)),
        out_shape=jax.ShapeDtypeStruct((_B, _N, _AB), jnp.float32),
        compiler_params=pltpu.CompilerParams(
            dimension_semantics=("parallel", "parallel"),
        ),
    )(y, nbr_r, mask_r, f_r,
      W1.astype(_BF), b1.reshape(1, _NF), W2.astype(_BF), b2.reshape(1, _NF),
      Wf.astype(_BF), bf.reshape(1, _AB),
      Wd.astype(_BF), bd.reshape(1, _AB))
    return out


# speedup vs baseline: 1.1847x; 1.1371x over previous
import jax
import jax.numpy as jnp
from jax.experimental import pallas as pl
from jax.experimental.pallas import tpu as pltpu

_B, _N, _NBH = 8, 512, 32
_AB, _SB, _NF = 256, 64, 256
_BN = 64
_NBLK = _N // _BN
_E = _BN * _NBH

_BF = jnp.bfloat16


def _ssp(v):
    return jnp.maximum(v, 0.0) + (jnp.log1p(jnp.exp(-jnp.abs(v))) - jnp.log(2.0))


def _y_kernel(x_ref, Wi_ref, y_ref):
    y_ref[0] = jnp.dot(x_ref[0].astype(_BF), Wi_ref[:],
                       preferred_element_type=jnp.float32).astype(_BF)


def _block_kernel(y_ref, nbr_ref, f_ref,
                  W1_ref, b1_ref, W2_ref, b2_ref,
                  Wf_ref, bf_ref, Wd_ref, bd_ref,
                  out_ref):
    f = f_ref[0].reshape(_E, _SB)
    h = _ssp(jnp.dot(f.astype(_BF), W1_ref[:],
                     preferred_element_type=jnp.float32) + b1_ref[:])
    wfilt = jnp.dot(h.astype(_BF), W2_ref[:],
                    preferred_element_type=jnp.float32) + b2_ref[:]

    idx = nbr_ref[0, 0]
    onehot = (idx == jax.lax.broadcasted_iota(jnp.int32, (_E, _N), 1)
              ).astype(_BF)
    y_nbh = jnp.dot(onehot, y_ref[0], preferred_element_type=jnp.float32)

    agg = (y_nbh * wfilt).reshape(_BN, _NBH, _NF).sum(axis=1)
    v = _ssp(jnp.dot(agg.astype(_BF), Wf_ref[:],
                     preferred_element_type=jnp.float32) + bf_ref[:])
    out_ref[0] = jnp.dot(v.astype(_BF), Wd_ref[:],
                         preferred_element_type=jnp.float32) + bd_ref[:]


def kernel(x, r_ij, neighbors, neighbor_mask, f_ij,
           W1, b1, W2, b2, Wi, Wf, bf, Wd, bd):
    del r_ij

    y = pl.pallas_call(
        _y_kernel,
        grid=(_B,),
        in_specs=[
            pl.BlockSpec((1, _N, _AB), lambda b: (b, 0, 0)),
            pl.BlockSpec((_AB, _NF), lambda b: (0, 0)),
        ],
        out_specs=pl.BlockSpec((1, _N, _NF), lambda b: (b, 0, 0)),
        out_shape=jax.ShapeDtypeStruct((_B, _N, _NF), _BF),
        compiler_params=pltpu.CompilerParams(
            dimension_semantics=("parallel",),
        ),
    )(x, Wi.astype(_BF))

    nbr_eff = jnp.where(neighbor_mask > 0, neighbors, -1)
    nbr_r = nbr_eff.reshape(_B, _NBLK, _E, 1)
    full = lambda shape: pl.BlockSpec(shape, lambda b, nb: (0,) * len(shape))

    out = pl.pallas_call(
        _block_kernel,
        grid=(_B, _NBLK),
        in_specs=[
            pl.BlockSpec((1, _N, _NF), lambda b, nb: (b, 0, 0)),
            pl.BlockSpec((1, 1, _E, 1), lambda b, nb: (b, nb, 0, 0)),
            pl.BlockSpec((1, _BN, _NBH, _SB), lambda b, nb: (b, nb, 0, 0)),
            full((_SB, _NF)),
            full((1, _NF)),
            full((_NF, _NF)),
            full((1, _NF)),
            full((_NF, _AB)),
            full((1, _AB)),
            full((_AB, _AB)),
            full((1, _AB)),
        ],
        out_specs=pl.BlockSpec((1, _BN, _AB), lambda b, nb: (b, nb, 0)),
        out_shape=jax.ShapeDtypeStruct((_B, _N, _AB), jnp.float32),
        compiler_params=pltpu.CompilerParams(
            dimension_semantics=("parallel", "parallel"),
        ),
    )(y, nbr_r, f_ij,
      W1.astype(_BF), b1.reshape(1, _NF), W2.astype(_BF), b2.reshape(1, _NF),
      Wf.astype(_BF), bf.reshape(1, _AB),
      Wd.astype(_BF), bd.reshape(1, _AB))
    return out


# speedup vs baseline: 1.3631x; 1.1506x over previous
import jax
import jax.numpy as jnp
from jax.experimental import pallas as pl
from jax.experimental.pallas import tpu as pltpu

_B, _N, _NBH = 8, 512, 32
_AB, _SB, _NF = 256, 64, 256
_BN = 256
_NBLK = _N // _BN
_E = _BN * _NBH

_BF = jnp.bfloat16


_LOG2E = 1.4426950408889634
_LN2 = 0.6931471805599453


def _ssp(v):
    t = jnp.exp2(jnp.abs(v) * -_LOG2E)
    return jnp.maximum(v, 0.0) + (jnp.log2(1.0 + t) - 1.0) * _LN2


def _y_kernel(x_ref, Wi_ref, y_ref):
    y_ref[0] = jnp.dot(x_ref[0].astype(_BF), Wi_ref[:],
                       preferred_element_type=jnp.float32).astype(_BF)


def _block_kernel(y_ref, nbr_ref, f_ref,
                  W1_ref, b1_ref, W2_ref, b2_ref,
                  Wf_ref, bf_ref, Wd_ref, bd_ref,
                  out_ref):
    f = f_ref[0].reshape(_E, _SB)
    h = _ssp(jnp.dot(f.astype(_BF), W1_ref[:],
                     preferred_element_type=jnp.float32) + b1_ref[:])
    wfilt = jnp.dot(h.astype(_BF), W2_ref[:],
                    preferred_element_type=jnp.float32) + b2_ref[:]

    idx = nbr_ref[0, 0]
    onehot = (idx == jax.lax.broadcasted_iota(jnp.int32, (_E, _N), 1)
              ).astype(_BF)
    y_nbh = jnp.dot(onehot, y_ref[0], preferred_element_type=jnp.float32)

    agg = (y_nbh * wfilt).reshape(_BN, _NBH, _NF).sum(axis=1)
    v = _ssp(jnp.dot(agg.astype(_BF), Wf_ref[:],
                     preferred_element_type=jnp.float32) + bf_ref[:])
    out_ref[0] = jnp.dot(v.astype(_BF), Wd_ref[:],
                         preferred_element_type=jnp.float32) + bd_ref[:]


def kernel(x, r_ij, neighbors, neighbor_mask, f_ij,
           W1, b1, W2, b2, Wi, Wf, bf, Wd, bd):
    del r_ij

    y = pl.pallas_call(
        _y_kernel,
        grid=(_B,),
        in_specs=[
            pl.BlockSpec((1, _N, _AB), lambda b: (b, 0, 0)),
            pl.BlockSpec((_AB, _NF), lambda b: (0, 0)),
        ],
        out_specs=pl.BlockSpec((1, _N, _NF), lambda b: (b, 0, 0)),
        out_shape=jax.ShapeDtypeStruct((_B, _N, _NF), _BF),
        compiler_params=pltpu.CompilerParams(
            dimension_semantics=("parallel",),
        ),
    )(x, Wi.astype(_BF))

    nbr_eff = jnp.where(neighbor_mask > 0, neighbors, -1)
    nbr_r = nbr_eff.reshape(_B, _NBLK, _E, 1)
    full = lambda shape: pl.BlockSpec(shape, lambda b, nb: (0,) * len(shape))

    out = pl.pallas_call(
        _block_kernel,
        grid=(_B, _NBLK),
        in_specs=[
            pl.BlockSpec((1, _N, _NF), lambda b, nb: (b, 0, 0)),
            pl.BlockSpec((1, 1, _E, 1), lambda b, nb: (b, nb, 0, 0)),
            pl.BlockSpec((1, _BN, _NBH, _SB), lambda b, nb: (b, nb, 0, 0)),
            full((_SB, _NF)),
            full((1, _NF)),
            full((_NF, _NF)),
            full((1, _NF)),
            full((_NF, _AB)),
            full((1, _AB)),
            full((_AB, _AB)),
            full((1, _AB)),
        ],
        out_specs=pl.BlockSpec((1, _BN, _AB), lambda b, nb: (b, nb, 0)),
        out_shape=jax.ShapeDtypeStruct((_B, _N, _AB), jnp.float32),
        compiler_params=pltpu.CompilerParams(
            dimension_semantics=("parallel", "parallel"),
        ),
    )(y, nbr_r, f_ij,
      W1.astype(_BF), b1.reshape(1, _NF), W2.astype(_BF), b2.reshape(1, _NF),
      Wf.astype(_BF), bf.reshape(1, _AB),
      Wd.astype(_BF), bd.reshape(1, _AB))
    return out
